# Initial kernel scaffold; baseline (speedup 1.0000x reference)
import jax
import jax.numpy as jnp
from jax.experimental import pallas as pl


def kernel(table, subtoken_ids, segment_ids):
    raise NotImplementedError("write your pallas kernel here")


# speedup vs baseline: 5.2859x; 5.2859x over previous
import jax
import jax.numpy as jnp
from jax import lax
from jax.experimental import pallas as pl
from jax.experimental.pallas import tpu as pltpu
from jax.experimental.pallas import tpu_sc as plsc

H = 64
NUM_NODES = 100000
S = 512
G = 128
NW = 32
NB = (NUM_NODES + S - 1) // S
NODES_PAD = NB * S
BPW = (NB + NW - 1) // NW
NSTARTS = ((NB + 1 + 15) // 16) * 16
ACC_ROWS = S + 8


def _body(table_h, ids_h, seg_h, starts_h, zeros_h, out_h,
          starts_v, idx_v, seg_v, lidx_v, rows_v, acc_sh, sem):
    cid = lax.axis_index("c")
    sid = lax.axis_index("s")
    wid = sid * 2 + cid
    arow = sid * ACC_ROWS

    pltpu.sync_copy(starts_h, starts_v)

    def block_body(i, carry):
        b = wid + i * NW

        @pl.when(b < NB)
        def _():
            pltpu.sync_copy(zeros_h, acc_sh.at[pl.ds(arow, ACC_ROWS)])

            base = pl.multiple_of((b // 8) * 8, 8)
            win = starts_v[pl.ds(base, 16)]
            r = b - base
            s0 = win[0]
            s1 = win[1]
            for l in range(1, 9):
                s0 = jnp.where(r == l, win[l], s0)
                s1 = jnp.where(r == l, win[l + 1], s1)
            a0 = s0 & ~7
            nchunks = (s1 - a0 + (G - 1)) // G
            bs = b * S

            def chunk(k, c2):
                off = pl.multiple_of(a0 + k * G, 8)
                pltpu.sync_copy(ids_h.at[pl.ds(off, G)], idx_v)
                pltpu.sync_copy(seg_h.at[pl.ds(off, G)], seg_v)
                for q in range(G // 16):
                    sv = seg_v[pl.ds(q * 16, 16)]
                    inb = (sv >= bs) & (sv < bs + S)
                    lidx_v[pl.ds(q * 16, 16)] = jnp.where(inb, sv - bs, S) + arow
                pltpu.async_copy(table_h.at[idx_v], rows_v, sem).wait()
                pltpu.sync_copy(rows_v, acc_sh.at[lidx_v], add=True)
                return c2

            lax.fori_loop(0, nchunks, chunk, 0)
            pltpu.sync_copy(acc_sh.at[pl.ds(arow, S)], out_h.at[pl.ds(bs, S)])

        return carry

    lax.fori_loop(0, BPW, block_body, 0)


def kernel(table, subtoken_ids, segment_ids):
    ids = subtoken_ids.astype(jnp.int32)
    seg = segment_ids.astype(jnp.int32)
    total = ids.shape[0]

    ids_p = jnp.concatenate([ids, jnp.zeros((G,), jnp.int32)])
    seg_p = jnp.concatenate([seg, jnp.full((G,), NODES_PAD, jnp.int32)])

    block_firsts = jnp.arange(NB + 1, dtype=jnp.int32) * S
    starts = jnp.searchsorted(seg, block_firsts, side="left").astype(jnp.int32)
    starts = jnp.concatenate(
        [starts, jnp.full((NSTARTS - (NB + 1),), total, jnp.int32)])

    zeros = jnp.zeros((ACC_ROWS, H), jnp.float32)

    kfn = pl.kernel(
        _body,
        out_type=jax.ShapeDtypeStruct((NODES_PAD, H), jnp.float32),
        mesh=plsc.VectorSubcoreMesh(core_axis_name="c", subcore_axis_name="s"),
        compiler_params=pltpu.CompilerParams(use_tc_tiling_on_sc=False),
        scratch_types=[
            pltpu.VMEM((NSTARTS,), jnp.int32),
            pltpu.VMEM((G,), jnp.int32),
            pltpu.VMEM((G,), jnp.int32),
            pltpu.VMEM((G,), jnp.int32),
            pltpu.VMEM((G, H), jnp.float32),
            pltpu.VMEM_SHARED((16 * ACC_ROWS, H), jnp.float32),
            pltpu.SemaphoreType.DMA,
        ],
    )
    out = kfn(table, ids_p, seg_p, starts, zeros)
    return out[:NUM_NODES]
